# baseline (speedup 1.0000x reference)
import jax
import jax.numpy as jnp
from jax.experimental import pallas as pl
from jax.experimental.pallas import tpu as pltpu

BATCH = 1024
FEAT = 64
BITS = 16
ENC = FEAT * BITS
HID = 2048
CLASSES = 10
SEG_THRESH = 4.0

BN = 256
NT = HID // BN
NT0 = ENC // BN

_RHS1 = (((1,), (1,)), ((), ()))


def _fused(x_ref, w0_ref, w1_ref, w2_ref, ow_ref, out_ref,
           a0_ref, a1_ref, a2_ref):
    l = pl.program_id(0)
    j = pl.program_id(1)

    @pl.when(jnp.logical_and(l == 0, j == 0))
    def _init():
        k = jnp.floor(x_ref[:] * (BITS - 1.0))
        jf = jax.lax.broadcasted_iota(jnp.int32, (FEAT, ENC), 1)
        ff = jax.lax.broadcasted_iota(jnp.int32, (FEAT, ENC), 0)
        expand = (jf // BITS == ff).astype(jnp.float32)
        kr = jnp.dot(k, expand, preferred_element_type=jnp.float32)
        t = (jax.lax.broadcasted_iota(jnp.int32, (1, ENC), 1) % BITS
             ).astype(jnp.float32)
        enc = (kr >= t).astype(jnp.float32)
        for jj in range(NT0):
            a0_ref[jj] = enc[:, jj * BN:(jj + 1) * BN]
        out_ref[:] = jnp.zeros_like(out_ref)

    def _stage(w_ref, src_ref, nsrc, dst_ref):
        z = jax.lax.dot_general(src_ref[0], w_ref[:, :BN], _RHS1,
                                preferred_element_type=jnp.float32)
        for jj in range(1, nsrc):
            z += jax.lax.dot_general(src_ref[jj],
                                     w_ref[:, jj * BN:(jj + 1) * BN], _RHS1,
                                     preferred_element_type=jnp.float32)
        act = (z >= SEG_THRESH).astype(jnp.float32)
        if dst_ref is not None:
            dst_ref[j] = act
        out_ref[:] += jnp.dot(act, ow_ref[0],
                              preferred_element_type=jnp.float32)

    @pl.when(l == 0)
    def _l0():
        _stage(w0_ref, a0_ref, NT0, a1_ref)

    @pl.when(l == 1)
    def _l1():
        _stage(w1_ref, a1_ref, NT, a2_ref)

    @pl.when(l == 2)
    def _l2():
        _stage(w2_ref, a2_ref, NT, None)


def kernel(x, W0, W1, W2, outW):
    grid = (3, NT)
    return pl.pallas_call(
        _fused,
        grid=grid,
        in_specs=[
            pl.BlockSpec((BATCH, FEAT), lambda l, j: (0, 0)),
            pl.BlockSpec((BN, ENC),
                         lambda l, j: (jnp.where(l == 0, j, NT - 1), 0)),
            pl.BlockSpec((BN, HID),
                         lambda l, j: (jnp.where(l < 1, 0,
                                                 jnp.where(l == 1, j, NT - 1)),
                                       0)),
            pl.BlockSpec((BN, HID),
                         lambda l, j: (jnp.where(l < 2, 0, j), 0)),
            pl.BlockSpec((1, BN, CLASSES), lambda l, j: (l, j, 0)),
        ],
        out_specs=pl.BlockSpec((BATCH, CLASSES), lambda l, j: (0, 0)),
        out_shape=jax.ShapeDtypeStruct((BATCH, CLASSES), jnp.float32),
        scratch_shapes=[
            pltpu.VMEM((NT0, BATCH, BN), jnp.float32),
            pltpu.VMEM((NT, BATCH, BN), jnp.float32),
            pltpu.VMEM((NT, BATCH, BN), jnp.float32),
        ],
    )(x, W0, W1, W2, outW)


# speedup vs baseline: 1.1271x; 1.1271x over previous
import jax
import jax.numpy as jnp
from jax.experimental import pallas as pl
from jax.experimental.pallas import tpu as pltpu

BATCH = 1024
FEAT = 64
BITS = 16
ENC = FEAT * BITS
HID = 2048
CLASSES = 10
SEG_THRESH = 4.0

BN = 512
NT = HID // BN
NT0 = ENC // BN

_RHS1 = (((1,), (1,)), ((), ()))


def _fused(x_ref, w0_ref, w1_ref, w2_ref, ow_ref, out_ref,
           a0_ref, a1_ref, a2_ref):
    l = pl.program_id(0)
    j = pl.program_id(1)

    @pl.when(jnp.logical_and(l == 0, j == 0))
    def _init():
        k = jnp.floor(x_ref[:] * (BITS - 1.0))
        jf = jax.lax.broadcasted_iota(jnp.int32, (FEAT, ENC), 1)
        ff = jax.lax.broadcasted_iota(jnp.int32, (FEAT, ENC), 0)
        expand = (jf // BITS == ff).astype(jnp.float32)
        kr = jnp.dot(k, expand, preferred_element_type=jnp.float32)
        t = (jax.lax.broadcasted_iota(jnp.int32, (1, ENC), 1) % BITS
             ).astype(jnp.float32)
        a0_ref[:] = (kr >= t).astype(jnp.float32)
        out_ref[:] = jnp.zeros_like(out_ref)

    def _stage(w_ref, src_ref, nsrc, dst_ref):
        z = jax.lax.dot_general(src_ref[:], w_ref[:], _RHS1,
                                preferred_element_type=jnp.float32)
        act = (z >= SEG_THRESH).astype(jnp.float32)
        if dst_ref is not None:
            dst_ref[:, pl.ds(j * BN, BN)] = act
        out_ref[:] += jnp.dot(act, ow_ref[0],
                              preferred_element_type=jnp.float32)

    @pl.when(l == 0)
    def _l0():
        _stage(w0_ref, a0_ref, NT0, a1_ref)

    @pl.when(l == 1)
    def _l1():
        _stage(w1_ref, a1_ref, NT, a2_ref)

    @pl.when(l == 2)
    def _l2():
        _stage(w2_ref, a2_ref, NT, None)


def kernel(x, W0, W1, W2, outW):
    grid = (3, NT)
    return pl.pallas_call(
        _fused,
        grid=grid,
        in_specs=[
            pl.BlockSpec((BATCH, FEAT), lambda l, j: (0, 0)),
            pl.BlockSpec((BN, ENC),
                         lambda l, j: (jnp.where(l == 0, j, NT - 1), 0)),
            pl.BlockSpec((BN, HID),
                         lambda l, j: (jnp.where(l < 1, 0,
                                                 jnp.where(l == 1, j, NT - 1)),
                                       0)),
            pl.BlockSpec((BN, HID),
                         lambda l, j: (jnp.where(l < 2, 0, j), 0)),
            pl.BlockSpec((1, BN, CLASSES), lambda l, j: (l, j, 0)),
        ],
        out_specs=pl.BlockSpec((BATCH, CLASSES), lambda l, j: (0, 0)),
        out_shape=jax.ShapeDtypeStruct((BATCH, CLASSES), jnp.float32),
        scratch_shapes=[
            pltpu.VMEM((BATCH, ENC), jnp.float32),
            pltpu.VMEM((BATCH, HID), jnp.float32),
            pltpu.VMEM((BATCH, HID), jnp.float32),
        ],
    )(x, W0, W1, W2, outW)
